# baseline (speedup 1.0000x reference)
import functools

import jax
import jax.numpy as jnp
from jax import lax
from jax.experimental import pallas as pl
from jax.experimental.pallas import tpu as pltpu
from jax.experimental.pallas import tpu_sc as plsc

_NC = 2
_NS = 16
_K = 64


@functools.partial(jax.jit, static_argnames=("n_nodes", "channels"))
def _spmm_partials(h, idx2d, zeros, *, n_nodes, channels):
    nrows_total = idx2d.shape[0]
    np_nodes = zeros.shape[0]
    ntiles = _NC * _NS
    nct = nrows_total // ntiles
    rpt = np_nodes // _NS
    mesh = plsc.VectorSubcoreMesh(core_axis_name="c", subcore_axis_name="s")

    nbuf = 2
    nh = 2
    nch = nct // nh
    @functools.partial(
        pl.kernel,
        out_type=jax.ShapeDtypeStruct((2 * np_nodes, channels), jnp.float32),
        mesh=mesh,
        scratch_types=(
            [pltpu.VMEM((nch, 2 * _K), jnp.int32)]
            + [pltpu.VMEM((_K, channels), jnp.float32)] * nbuf
            + [pltpu.VMEM_SHARED((np_nodes, channels), jnp.float32)]
            + [pltpu.SemaphoreType.DMA] * (2 * nbuf)
        ),
    )
    def spmm(h_hbm, idx_hbm, zeros_hbm, out_hbm, idx_v, *scratch):
        rows = scratch[:nbuf]
        acc = scratch[nbuf]
        sem_g = scratch[nbuf + 1:nbuf + 1 + nbuf]
        sem_s = scratch[nbuf + 1 + nbuf:]
        cid = lax.axis_index("c")
        sid = lax.axis_index("s")
        tid = cid * _NS + sid

        def src_at(j):
            return idx_v.at[j, pl.ds(0, _K)]

        def dst_at(j):
            return idx_v.at[j, pl.ds(_K, _K)]

        def body(i, carry):
            base = i * nbuf
            for b in range(nbuf):
                j = base + b
                pltpu.make_async_copy(h_hbm.at[src_at(j)],
                                      rows[b], sem_g[b]).wait()
                pltpu.async_copy(rows[b], acc.at[dst_at(j)], sem_s[b],
                                 add=True)
                pltpu.make_async_copy(rows[b], acc.at[dst_at(j)],
                                      sem_s[b]).wait()
            for b in range(nbuf):
                j = base + b

                def _refill(b=b, j=j):
                    pltpu.async_copy(h_hbm.at[src_at(j + nbuf)],
                                     rows[b], sem_g[b])

                pl.when(j + nbuf < nch)(_refill)
            return carry

        for h in range(nh):
            pltpu.sync_copy(idx_hbm.at[pl.ds(tid * nct + h * nch, nch)], idx_v)
            for b in range(nbuf):
                pltpu.async_copy(h_hbm.at[src_at(b)], rows[b], sem_g[b])
            if h == 0:
                pltpu.sync_copy(zeros_hbm.at[pl.ds(sid * rpt, rpt)],
                                acc.at[pl.ds(sid * rpt, rpt)])
                plsc.subcore_barrier()
            lax.fori_loop(0, nch // nbuf, body, 0)

        plsc.subcore_barrier()
        pltpu.sync_copy(acc.at[pl.ds(sid * rpt, rpt)],
                        out_hbm.at[pl.ds(cid * np_nodes + sid * rpt, rpt)])

    return spmm(h, idx2d, zeros)


def _mm2_body(p0, p1, w0, w1, o):
    s = p0[...] + p1[...]
    hid = jnp.maximum(jnp.dot(s, w0[...], preferred_element_type=jnp.float32), 0.0)
    o[...] = jnp.dot(hid, w1[...], preferred_element_type=jnp.float32)


def _relu_body(q0, q1, o):
    o[...] = jnp.maximum(q0[...] + q1[...], 0.0)


def _mm_softmax_body(r0, r1, w2, o):
    s = jnp.dot(r0[...] + r1[...], w2[...], preferred_element_type=jnp.float32)
    m = jnp.max(s, axis=-1, keepdims=True)
    e = jnp.exp(s - m)
    o[...] = e / jnp.sum(e, axis=-1, keepdims=True)


def kernel(x, edge_index, W0, W1, W2):
    n, d_feat = x.shape
    e = edge_index.shape[1]
    c0 = W0.shape[1]
    c1 = W1.shape[1]
    ncls = W2.shape[1]

    npad = ((n + 127) // 128) * 128
    ntiles = _NC * _NS
    nct = -(-e // (ntiles * _K * 8)) * 8
    epad = ntiles * nct * _K
    src_p = jnp.concatenate(
        [edge_index[0], jnp.zeros((epad - e,), jnp.int32)])
    dst_p = jnp.concatenate(
        [edge_index[1], jnp.full((epad - e,), npad - 1, jnp.int32)])
    src2d = src_p.reshape(epad // _K, _K)
    dst2d = dst_p.reshape(epad // _K, _K)
    idx2d = jnp.concatenate([src2d, dst2d], axis=1)
    zeros_f = jnp.zeros((npad, d_feat), jnp.float32)

    blk = 1000
    grid = (n // blk,)

    agg0 = _spmm_partials(x, idx2d, zeros_f,
                          n_nodes=n, channels=d_feat)

    t1 = pl.pallas_call(
        _mm2_body,
        grid=grid,
        in_specs=[
            pl.BlockSpec((blk, d_feat), lambda i: (i, 0)),
            pl.BlockSpec((blk, d_feat), lambda i: (i, 0)),
            pl.BlockSpec((d_feat, c0), lambda i: (0, 0)),
            pl.BlockSpec((c0, c1), lambda i: (0, 0)),
        ],
        out_specs=pl.BlockSpec((blk, c1), lambda i: (i, 0)),
        out_shape=jax.ShapeDtypeStruct((n, c1), jnp.float32),
    )(agg0[:n], agg0[npad:npad + n], W0, W1)

    agg1 = _spmm_partials(t1, idx2d, zeros_f,
                          n_nodes=n, channels=c1)

    h1 = pl.pallas_call(
        _relu_body,
        grid=grid,
        in_specs=[
            pl.BlockSpec((blk, c1), lambda i: (i, 0)),
            pl.BlockSpec((blk, c1), lambda i: (i, 0)),
        ],
        out_specs=pl.BlockSpec((blk, c1), lambda i: (i, 0)),
        out_shape=jax.ShapeDtypeStruct((n, c1), jnp.float32),
    )(agg1[:n], agg1[npad:npad + n])

    agg2 = _spmm_partials(h1, idx2d, zeros_f,
                          n_nodes=n, channels=c1)

    out = pl.pallas_call(
        _mm_softmax_body,
        grid=grid,
        in_specs=[
            pl.BlockSpec((blk, c1), lambda i: (i, 0)),
            pl.BlockSpec((blk, c1), lambda i: (i, 0)),
            pl.BlockSpec((c1, ncls), lambda i: (0, 0)),
        ],
        out_specs=pl.BlockSpec((blk, ncls), lambda i: (i, 0)),
        out_shape=jax.ShapeDtypeStruct((n, ncls), jnp.float32),
    )(agg2[:n], agg2[npad:npad + n], W2)

    return out


# speedup vs baseline: 1.0745x; 1.0745x over previous
import functools

import jax
import jax.numpy as jnp
from jax import lax
from jax.experimental import pallas as pl
from jax.experimental.pallas import tpu as pltpu
from jax.experimental.pallas import tpu_sc as plsc

_NC = 2
_NS = 16
_K = 64


@functools.partial(jax.jit, static_argnames=("n_nodes", "channels"))
def _spmm_partials(h, idx2d, zeros, *, n_nodes, channels):
    nrows_total = idx2d.shape[0]
    np_nodes = zeros.shape[0]
    ntiles = _NC * _NS
    nct = nrows_total // ntiles
    rpt = np_nodes // _NS
    mesh = plsc.VectorSubcoreMesh(core_axis_name="c", subcore_axis_name="s")

    nbuf = 2
    nh = 2
    nch = nct // nh
    @functools.partial(
        pl.kernel,
        out_type=jax.ShapeDtypeStruct((2 * np_nodes, channels), jnp.float32),
        mesh=mesh,
        scratch_types=(
            [pltpu.VMEM((nch, 2 * _K), jnp.int32)]
            + [pltpu.VMEM((_K, channels), jnp.float32)] * nbuf
            + [pltpu.VMEM_SHARED((np_nodes, channels), jnp.float32)]
            + [pltpu.SemaphoreType.DMA] * (2 * nbuf)
        ),
    )
    def spmm(h_hbm, idx_hbm, zeros_hbm, out_hbm, idx_v, *scratch):
        rows = scratch[:nbuf]
        acc = scratch[nbuf]
        sem_g = scratch[nbuf + 1:nbuf + 1 + nbuf]
        sem_s = scratch[nbuf + 1 + nbuf:]
        cid = lax.axis_index("c")
        sid = lax.axis_index("s")
        tid = cid * _NS + sid

        def src_at(j):
            return idx_v.at[j, pl.ds(0, _K)]

        def dst_at(j):
            return idx_v.at[j, pl.ds(_K, _K)]

        def body(i, carry):
            base = i * nbuf
            for b in range(nbuf):
                j = base + b
                pltpu.make_async_copy(h_hbm.at[src_at(j)],
                                      rows[b], sem_g[b]).wait()
                pltpu.async_copy(rows[b], acc.at[dst_at(j)], sem_s[b],
                                 add=True)
            for b in range(nbuf):
                j = base + b
                pltpu.make_async_copy(rows[b], acc.at[dst_at(j)],
                                      sem_s[b]).wait()

                def _refill(b=b, j=j):
                    pltpu.async_copy(h_hbm.at[src_at(j + nbuf)],
                                     rows[b], sem_g[b])

                pl.when(j + nbuf < nch)(_refill)
            return carry

        for h in range(nh):
            pltpu.sync_copy(idx_hbm.at[pl.ds(tid * nct + h * nch, nch)], idx_v)
            for b in range(nbuf):
                pltpu.async_copy(h_hbm.at[src_at(b)], rows[b], sem_g[b])
            if h == 0:
                pltpu.sync_copy(zeros_hbm.at[pl.ds(sid * rpt, rpt)],
                                acc.at[pl.ds(sid * rpt, rpt)])
                plsc.subcore_barrier()
            lax.fori_loop(0, nch // nbuf, body, 0)

        plsc.subcore_barrier()
        pltpu.sync_copy(acc.at[pl.ds(sid * rpt, rpt)],
                        out_hbm.at[pl.ds(cid * np_nodes + sid * rpt, rpt)])

    return spmm(h, idx2d, zeros)


def _mm2_body(p0, p1, w0, w1, o):
    s = p0[...] + p1[...]
    hid = jnp.maximum(jnp.dot(s, w0[...], preferred_element_type=jnp.float32), 0.0)
    o[...] = jnp.dot(hid, w1[...], preferred_element_type=jnp.float32)


def _relu_body(q0, q1, o):
    o[...] = jnp.maximum(q0[...] + q1[...], 0.0)


def _mm_softmax_body(r0, r1, w2, o):
    s = jnp.dot(r0[...] + r1[...], w2[...], preferred_element_type=jnp.float32)
    m = jnp.max(s, axis=-1, keepdims=True)
    e = jnp.exp(s - m)
    o[...] = e / jnp.sum(e, axis=-1, keepdims=True)


def kernel(x, edge_index, W0, W1, W2):
    n, d_feat = x.shape
    e = edge_index.shape[1]
    c0 = W0.shape[1]
    c1 = W1.shape[1]
    ncls = W2.shape[1]

    npad = ((n + 127) // 128) * 128
    ntiles = _NC * _NS
    nct = -(-e // (ntiles * _K * 8)) * 8
    epad = ntiles * nct * _K
    src_p = jnp.concatenate(
        [edge_index[0], jnp.zeros((epad - e,), jnp.int32)])
    dst_p = jnp.concatenate(
        [edge_index[1], jnp.full((epad - e,), npad - 1, jnp.int32)])
    src2d = src_p.reshape(epad // _K, _K)
    dst2d = dst_p.reshape(epad // _K, _K)
    idx2d = jnp.concatenate([src2d, dst2d], axis=1)
    zeros_f = jnp.zeros((npad, d_feat), jnp.float32)

    blk = 1000
    grid = (n // blk,)

    agg0 = _spmm_partials(x, idx2d, zeros_f,
                          n_nodes=n, channels=d_feat)

    t1 = pl.pallas_call(
        _mm2_body,
        grid=grid,
        in_specs=[
            pl.BlockSpec((blk, d_feat), lambda i: (i, 0)),
            pl.BlockSpec((blk, d_feat), lambda i: (i, 0)),
            pl.BlockSpec((d_feat, c0), lambda i: (0, 0)),
            pl.BlockSpec((c0, c1), lambda i: (0, 0)),
        ],
        out_specs=pl.BlockSpec((blk, c1), lambda i: (i, 0)),
        out_shape=jax.ShapeDtypeStruct((n, c1), jnp.float32),
    )(agg0[:n], agg0[npad:npad + n], W0, W1)

    agg1 = _spmm_partials(t1, idx2d, zeros_f,
                          n_nodes=n, channels=c1)

    h1 = pl.pallas_call(
        _relu_body,
        grid=grid,
        in_specs=[
            pl.BlockSpec((blk, c1), lambda i: (i, 0)),
            pl.BlockSpec((blk, c1), lambda i: (i, 0)),
        ],
        out_specs=pl.BlockSpec((blk, c1), lambda i: (i, 0)),
        out_shape=jax.ShapeDtypeStruct((n, c1), jnp.float32),
    )(agg1[:n], agg1[npad:npad + n])

    agg2 = _spmm_partials(h1, idx2d, zeros_f,
                          n_nodes=n, channels=c1)

    out = pl.pallas_call(
        _mm_softmax_body,
        grid=grid,
        in_specs=[
            pl.BlockSpec((blk, c1), lambda i: (i, 0)),
            pl.BlockSpec((blk, c1), lambda i: (i, 0)),
            pl.BlockSpec((c1, ncls), lambda i: (0, 0)),
        ],
        out_specs=pl.BlockSpec((blk, ncls), lambda i: (i, 0)),
        out_shape=jax.ShapeDtypeStruct((n, ncls), jnp.float32),
    )(agg2[:n], agg2[npad:npad + n], W2)

    return out


# speedup vs baseline: 1.1545x; 1.0745x over previous
import functools

import jax
import jax.numpy as jnp
from jax import lax
from jax.experimental import pallas as pl
from jax.experimental.pallas import tpu as pltpu
from jax.experimental.pallas import tpu_sc as plsc

_NC = 2
_NS = 16
_K = 128


@functools.partial(jax.jit, static_argnames=("n_nodes", "channels"))
def _spmm_partials(h, idx2d, zeros, *, n_nodes, channels):
    nrows_total = idx2d.shape[0] // 2
    np_nodes = zeros.shape[0]
    ntiles = _NC * _NS
    nct = nrows_total // ntiles
    rpt = np_nodes // _NS
    mesh = plsc.VectorSubcoreMesh(core_axis_name="c", subcore_axis_name="s")

    nbuf = 2
    nh = 2
    nch = nct // nh
    @functools.partial(
        pl.kernel,
        out_type=jax.ShapeDtypeStruct((2 * np_nodes, channels), jnp.float32),
        mesh=mesh,
        scratch_types=(
            [pltpu.VMEM((nch, _K), jnp.int32)] * 2
            + [pltpu.VMEM((_K, channels), jnp.float32)] * nbuf
            + [pltpu.VMEM_SHARED((np_nodes, channels), jnp.float32)]
            + [pltpu.SemaphoreType.DMA] * (nbuf + 1)
        ),
    )
    def spmm(h_hbm, idx_hbm, zeros_hbm, out_hbm, src_v, dst_v,
             r0, r1, acc, sg0, sg1, sem_s):
        rows = (r0, r1)
        sem_g = (sg0, sg1)
        cid = lax.axis_index("c")
        sid = lax.axis_index("s")
        tid = cid * _NS + sid

        def gather(j):
            pltpu.async_copy(h_hbm.at[src_v.at[j]], rows[j % nbuf],
                             sem_g[j % nbuf])

        def gwait(j):
            pltpu.make_async_copy(h_hbm.at[src_v.at[j]], rows[j % nbuf],
                                  sem_g[j % nbuf]).wait()

        def scat(j):
            pltpu.async_copy(rows[j % nbuf], acc.at[dst_v.at[j]], sem_s,
                             add=True)

        def swait(j):
            pltpu.make_async_copy(rows[j % nbuf], acc.at[dst_v.at[j]],
                                  sem_s).wait()

        for h in range(nh):
            base = tid * nct + h * nch
            pltpu.sync_copy(idx_hbm.at[pl.ds(base, nch)], src_v)
            pltpu.sync_copy(idx_hbm.at[pl.ds(nrows_total + base, nch)], dst_v)
            gather(0)
            gather(1)
            if h == 0:
                pltpu.sync_copy(zeros_hbm.at[pl.ds(sid * rpt, rpt)],
                                acc.at[pl.ds(sid * rpt, rpt)])
                plsc.subcore_barrier()
            for j in range(nch):
                gwait(j)
                if j > 0:
                    swait(j - 1)
                    if j + 1 < nch:
                        gather(j + 1)
                scat(j)
            swait(nch - 1)

        plsc.subcore_barrier()
        pltpu.sync_copy(acc.at[pl.ds(sid * rpt, rpt)],
                        out_hbm.at[pl.ds(cid * np_nodes + sid * rpt, rpt)])

    return spmm(h, idx2d, zeros)


def _mm2_body(p0, p1, w0, w1, o):
    s = p0[...] + p1[...]
    hid = jnp.maximum(jnp.dot(s, w0[...], preferred_element_type=jnp.float32), 0.0)
    o[...] = jnp.dot(hid, w1[...], preferred_element_type=jnp.float32)


def _relu_body(q0, q1, o):
    o[...] = jnp.maximum(q0[...] + q1[...], 0.0)


def _mm_softmax_body(r0, r1, w2, o):
    s = jnp.dot(r0[...] + r1[...], w2[...], preferred_element_type=jnp.float32)
    m = jnp.max(s, axis=-1, keepdims=True)
    e = jnp.exp(s - m)
    o[...] = e / jnp.sum(e, axis=-1, keepdims=True)


def kernel(x, edge_index, W0, W1, W2):
    n, d_feat = x.shape
    e = edge_index.shape[1]
    c0 = W0.shape[1]
    c1 = W1.shape[1]
    ncls = W2.shape[1]

    npad = ((n + 127) // 128) * 128
    ntiles = _NC * _NS
    nct = -(-e // (ntiles * _K * 16)) * 16
    epad = ntiles * nct * _K
    src_p = jnp.concatenate(
        [edge_index[0], jnp.zeros((epad - e,), jnp.int32)])
    dst_p = jnp.concatenate(
        [edge_index[1], jnp.full((epad - e,), npad - 1, jnp.int32)])
    idx2d = jnp.concatenate(
        [src_p.reshape(epad // _K, _K), dst_p.reshape(epad // _K, _K)], axis=0)
    zeros_f = jnp.zeros((npad, d_feat), jnp.float32)

    blk = 1000
    grid = (n // blk,)

    agg0 = _spmm_partials(x, idx2d, zeros_f,
                          n_nodes=n, channels=d_feat)

    t1 = pl.pallas_call(
        _mm2_body,
        grid=grid,
        in_specs=[
            pl.BlockSpec((blk, d_feat), lambda i: (i, 0)),
            pl.BlockSpec((blk, d_feat), lambda i: (i, 0)),
            pl.BlockSpec((d_feat, c0), lambda i: (0, 0)),
            pl.BlockSpec((c0, c1), lambda i: (0, 0)),
        ],
        out_specs=pl.BlockSpec((blk, c1), lambda i: (i, 0)),
        out_shape=jax.ShapeDtypeStruct((n, c1), jnp.float32),
    )(agg0[:n], agg0[npad:npad + n], W0, W1)

    agg1 = _spmm_partials(t1, idx2d, zeros_f,
                          n_nodes=n, channels=c1)

    h1 = pl.pallas_call(
        _relu_body,
        grid=grid,
        in_specs=[
            pl.BlockSpec((blk, c1), lambda i: (i, 0)),
            pl.BlockSpec((blk, c1), lambda i: (i, 0)),
        ],
        out_specs=pl.BlockSpec((blk, c1), lambda i: (i, 0)),
        out_shape=jax.ShapeDtypeStruct((n, c1), jnp.float32),
    )(agg1[:n], agg1[npad:npad + n])

    agg2 = _spmm_partials(h1, idx2d, zeros_f,
                          n_nodes=n, channels=c1)

    out = pl.pallas_call(
        _mm_softmax_body,
        grid=grid,
        in_specs=[
            pl.BlockSpec((blk, c1), lambda i: (i, 0)),
            pl.BlockSpec((blk, c1), lambda i: (i, 0)),
            pl.BlockSpec((c1, ncls), lambda i: (0, 0)),
        ],
        out_specs=pl.BlockSpec((blk, ncls), lambda i: (i, 0)),
        out_shape=jax.ShapeDtypeStruct((n, ncls), jnp.float32),
    )(agg2[:n], agg2[npad:npad + n], W2)

    return out
